# Initial kernel scaffold; baseline (speedup 1.0000x reference)
import jax
import jax.numpy as jnp
from jax.experimental import pallas as pl


def kernel(X, H, hyperedge_weight, gamma0, beta0, W0, b0, wg0, bg0, gamma1, beta1, W1, b1, wg1, bg1, gamma2, beta2, W2, b2, wg2, bg2):
    raise NotImplementedError("write your pallas kernel here")


# speedup vs baseline: 8.3236x; 8.3236x over previous
import functools

import jax
import jax.numpy as jnp
from jax import lax
from jax.experimental import pallas as pl
from jax.experimental.pallas import tpu as pltpu
from jax.experimental.pallas import tpu_sc as plsc

N_NODES = 10000
N_INC = 320000
D_FEAT = 128
N_HEDGES = 10000

NC = 2
NS = 16
NW = NC * NS
CH = 128
NCHUNKS = N_INC // CH
KMAX = -(-NCHUNKS // NW)
EXTRA = NCHUNKS - (KMAX - 1) * NW
NPAD = 10240
ROWS_PER_TILE = NPAD // NS

_f32 = jnp.float32
_i32 = jnp.int32

_MESH = plsc.VectorSubcoreMesh(
    core_axis_name="c", subcore_axis_name="s", num_cores=NC, num_subcores=NS)


def _make_sc_hop(feat):
    @functools.partial(
        pl.kernel,
        mesh=_MESH,
        out_type=jax.ShapeDtypeStruct((NC, NPAD, feat), _f32),
        compiler_params=pltpu.CompilerParams(use_tc_tiling_on_sc=False),
        scratch_types=[
            pltpu.VMEM((1, CH), _i32),
            pltpu.VMEM((1, CH), _i32),
            pltpu.VMEM((CH, feat), _f32),
            pltpu.VMEM_SHARED((NPAD, feat), _f32),
            pltpu.SemaphoreType.DMA,
        ],
    )
    def sc_hop(gidx_hbm, widx_hbm, table_hbm, zeros_hbm, out_hbm,
               gi_v, wi_v, rows_v, acc_sh, sem):
        cid = lax.axis_index("c")
        sid = lax.axis_index("s")
        wid = sid * NC + cid

        pltpu.sync_copy(zeros_hbm.at[pl.ds(sid * ROWS_PER_TILE, ROWS_PER_TILE)],
                        acc_sh.at[pl.ds(sid * ROWS_PER_TILE, ROWS_PER_TILE)])
        plsc.subcore_barrier()

        nk = jnp.where(wid < EXTRA, KMAX, KMAX - 1)

        def body(k, carry):
            base = (wid + k * NW) * CH
            pltpu.sync_copy(gidx_hbm.at[pl.ds(base, CH)], gi_v.at[0])
            pltpu.sync_copy(widx_hbm.at[pl.ds(base, CH)], wi_v.at[0])
            pltpu.async_copy(table_hbm.at[gi_v.at[0]], rows_v, sem).wait()
            pltpu.sync_copy(rows_v, acc_sh.at[wi_v.at[0]], add=True)
            return carry

        lax.fori_loop(0, nk, body, 0)
        plsc.subcore_barrier()
        pltpu.sync_copy(acc_sh.at[pl.ds(sid * ROWS_PER_TILE, ROWS_PER_TILE)],
                        out_hbm.at[cid, pl.ds(sid * ROWS_PER_TILE, ROWS_PER_TILE)])

    return sc_hop


_sc_hop_feat = _make_sc_hop(D_FEAT)


@functools.partial(
    pl.kernel,
    mesh=_MESH,
    out_type=(jax.ShapeDtypeStruct((NC, NPAD, 16), _f32),
              jax.ShapeDtypeStruct((NC, NPAD, 16), _f32)),
    compiler_params=pltpu.CompilerParams(use_tc_tiling_on_sc=False),
    scratch_types=[
        pltpu.VMEM((1, CH), _i32),
        pltpu.VMEM((1, CH), _i32),
        pltpu.VMEM((CH, 16), _f32),
        pltpu.VMEM_SHARED((NPAD, 16), _f32),
        pltpu.VMEM_SHARED((NPAD, 16), _f32),
        pltpu.SemaphoreType.DMA,
    ],
)
def _sc_prep(src_hbm, dst_hbm, aug_hbm, zeros_hbm, outD_hbm, outB_hbm,
             si_v, di_v, rows_v, accD_sh, accB_sh, sem):
    cid = lax.axis_index("c")
    sid = lax.axis_index("s")
    wid = sid * NC + cid

    pltpu.sync_copy(zeros_hbm.at[pl.ds(sid * ROWS_PER_TILE, ROWS_PER_TILE)],
                    accD_sh.at[pl.ds(sid * ROWS_PER_TILE, ROWS_PER_TILE)])
    pltpu.sync_copy(zeros_hbm.at[pl.ds(sid * ROWS_PER_TILE, ROWS_PER_TILE)],
                    accB_sh.at[pl.ds(sid * ROWS_PER_TILE, ROWS_PER_TILE)])
    plsc.subcore_barrier()

    nk = jnp.where(wid < EXTRA, KMAX, KMAX - 1)

    def body(k, carry):
        base = (wid + k * NW) * CH
        pltpu.sync_copy(src_hbm.at[pl.ds(base, CH)], si_v.at[0])
        pltpu.sync_copy(dst_hbm.at[pl.ds(base, CH)], di_v.at[0])
        pltpu.async_copy(aug_hbm.at[di_v.at[0]], rows_v, sem).wait()
        pltpu.sync_copy(rows_v, accD_sh.at[si_v.at[0]], add=True)
        pltpu.sync_copy(rows_v, accB_sh.at[di_v.at[0]], add=True)
        return carry

    lax.fori_loop(0, nk, body, 0)
    plsc.subcore_barrier()
    pltpu.sync_copy(accD_sh.at[pl.ds(sid * ROWS_PER_TILE, ROWS_PER_TILE)],
                    outD_hbm.at[cid, pl.ds(sid * ROWS_PER_TILE, ROWS_PER_TILE)])
    pltpu.sync_copy(accB_sh.at[pl.ds(sid * ROWS_PER_TILE, ROWS_PER_TILE)],
                    outB_hbm.at[cid, pl.ds(sid * ROWS_PER_TILE, ROWS_PER_TILE)])


def _bn_mm_gate_body(x_ref, g_ref, be_ref, w_ref, wg_ref, bg_ref,
                     xl_ref, gate_ref):
    x = x_ref[...]
    mu = jnp.mean(x, axis=0, keepdims=True)
    xc = x - mu
    var = jnp.mean(xc * xc, axis=0, keepdims=True)
    xn = xc * lax.rsqrt(var + 1e-5) * g_ref[...] + be_ref[...]
    xl_ref[...] = lax.dot_general(
        xn, w_ref[...], (((1,), (1,)), ((), ())),
        preferred_element_type=_f32)
    z = jnp.sum(x * wg_ref[...], axis=1, keepdims=True) + bg_ref[...]
    gate_ref[...] = 1.0 / (1.0 + jnp.exp(-z))


_tc_bn_mm_gate = pl.pallas_call(
    _bn_mm_gate_body,
    out_shape=(jax.ShapeDtypeStruct((N_NODES, D_FEAT), _f32),
               jax.ShapeDtypeStruct((N_NODES, 1), _f32)),
)


def _safe_inv(d):
    return jnp.where(d == 0, 0.0, 1.0 / jnp.where(d == 0, 1.0, d))


def _merge_he_body(p_ref, accB_ref, he_ref):
    s = p_ref[0] + p_ref[1]
    b = accB_ref[0, :, 1:2] + accB_ref[1, :, 1:2]
    he_ref[...] = s * _safe_inv(b)


_tc_merge_he = pl.pallas_call(
    _merge_he_body,
    out_shape=jax.ShapeDtypeStruct((N_NODES, D_FEAT), _f32),
)


def _update_body(q_ref, accD_ref, b_ref, gate_ref, x_ref, out_ref):
    s = q_ref[0] + q_ref[1]
    d = accD_ref[0, :, 0:1] + accD_ref[1, :, 0:1]
    h = jnp.maximum(s * _safe_inv(d) + b_ref[...], 0.0)
    out_ref[...] = x_ref[...] + h * gate_ref[...]


_tc_update = pl.pallas_call(
    _update_body,
    out_shape=jax.ShapeDtypeStruct((N_NODES, D_FEAT), _f32),
)


def _update_final_body(q_ref, accD_ref, b_ref, gate_ref, x_ref, x0_ref,
                       out_ref):
    s = q_ref[0] + q_ref[1]
    d = accD_ref[0, :, 0:1] + accD_ref[1, :, 0:1]
    h = jnp.maximum(s * _safe_inv(d) + b_ref[...], 0.0)
    xn = x_ref[...] + h * gate_ref[...]
    out_ref[...] = 2.0 * xn + x0_ref[...]


_tc_update_final = pl.pallas_call(
    _update_final_body,
    out_shape=jax.ShapeDtypeStruct((N_NODES, D_FEAT), _f32),
)


def kernel(X, H, hyperedge_weight,
           gamma0, beta0, W0, b0, wg0, bg0,
           gamma1, beta1, W1, b1, wg1, bg1,
           gamma2, beta2, W2, b2, wg2, bg2):
    src = H[0].astype(_i32)
    dst = H[1].astype(_i32)
    hw = hyperedge_weight.astype(_f32)
    aug = jnp.concatenate(
        [hw[:, None], jnp.ones((N_HEDGES, 1), _f32),
         jnp.zeros((N_HEDGES, 14), _f32)], axis=1)
    zeros16 = jnp.zeros((NPAD, 16), _f32)
    zeros128 = jnp.zeros((NPAD, D_FEAT), _f32)

    accD, accB = _sc_prep(src, dst, aug, zeros16)
    accD = accD[:, :N_NODES]
    accB = accB[:, :N_NODES]

    params = [
        (gamma0, beta0, W0, b0, wg0, bg0),
        (gamma1, beta1, W1, b1, wg1, bg1),
        (gamma2, beta2, W2, b2, wg2, bg2),
    ]
    x0 = X
    x = X
    for layer, (g, be, W, b, wg, bg) in enumerate(params):
        xl, gate = _tc_bn_mm_gate(x, g.reshape(1, -1), be.reshape(1, -1),
                                  W, wg, bg.reshape(1, 1))
        p = _sc_hop_feat(src, dst, xl, zeros128)[:, :N_NODES]
        he = _tc_merge_he(p, accB)
        q = _sc_hop_feat(dst, src, he, zeros128)[:, :N_NODES]
        if layer < 2:
            x = _tc_update(q, accD, b.reshape(1, -1), gate, x)
        else:
            x = _tc_update_final(q, accD, b.reshape(1, -1), gate, x, x0)
    return x
